# baseline (speedup 1.0000x reference)
import functools

import jax
import jax.numpy as jnp
from jax import lax
from jax.experimental import pallas as pl
from jax.experimental.pallas import tpu as pltpu
from jax.experimental.pallas import tpu_sc as plsc

_N = 256
_DT = 0.05
_DEPTH = 8
_BATCH = 262144
_L = 16
_NC = 2
_NS = 16
_NW = _NC * _NS
_BPW = _BATCH // _NW
_UNROLL = 2
_ITERS = _BPW // (_L * _UNROLL)

_INV_STEP = 2.56
_STEP = 0.390625
_DTB = 0.050048828125


def _rnbf16(a):
    u = plsc.bitcast(a, jnp.int32)
    u = (u + 0x7FFF + ((u >> 16) & 1)) & jnp.int32(-65536)
    return plsc.bitcast(u, jnp.float32)


def _nearest_idx(x):
    return (x * jnp.float32(_N) + jnp.float32(0.5)).astype(jnp.int32)


def _sc_body(x_hbm, f_hbm, out_hbm, buf, ftab):
    wid = lax.axis_index("s") * _NC + lax.axis_index("c")
    base = wid * (_BPW * 2)
    pltpu.sync_copy(f_hbm, ftab.at[pl.ds(0, _N)])
    pltpu.sync_copy(x_hbm.at[pl.ds(base, _BPW * 2)], buf)
    last = plsc.load_gather(ftab, [jnp.full((_L,), _N - 1, jnp.int32)])
    for j in range(_N // _L):
        ftab[pl.ds(_N + j * _L, _L)] = last

    dt = jnp.float32(_DT)

    @plsc.parallel_loop(0, _BPW // _L, unroll=_UNROLL)
    def _loop(i):
        off = (i >> 3) * 256 + (i & 7) * _L
        x = buf[pl.ds(off, _L)]
        v = buf[pl.ds(off + 128, _L)]
        v = _rnbf16(v)
        x = _rnbf16(x) + v * jnp.float32(_DTB)
        for _ in range(_DEPTH):
            idx = _nearest_idx(x)
            f = plsc.load_gather(ftab, [idx])
            v = v + dt * f
            x = x + v * dt
        buf[pl.ds(off, _L)] = x
        buf[pl.ds(off + 128, _L)] = v

    pltpu.sync_copy(buf, out_hbm.at[pl.ds(base, _BPW * 2)])


_mesh = plsc.VectorSubcoreMesh(core_axis_name="c", subcore_axis_name="s")

_sc_kernel = functools.partial(
    pl.kernel,
    mesh=_mesh,
    out_type=jax.ShapeDtypeStruct((_BATCH * 2,), jnp.float32),
    scratch_types=[
        pltpu.VMEM((_BPW * 2,), jnp.float32),
        pltpu.VMEM((2 * _N,), jnp.float32),
    ],
    compiler_params=pltpu.CompilerParams(needs_layout_passes=False),
)(_sc_body)


@jax.jit
def kernel(X, force):
    xf = X.reshape(_BATCH // 128, 128, 2).transpose(0, 2, 1).reshape(-1)
    out = _sc_kernel(xf, force)
    return (out.reshape(_BATCH // 128, 2, 128)
               .transpose(0, 2, 1).reshape(_BATCH, 2))


# speedup vs baseline: 1.0032x; 1.0032x over previous
import functools

import jax
import jax.numpy as jnp
from jax import lax
from jax.experimental import pallas as pl
from jax.experimental.pallas import tpu as pltpu
from jax.experimental.pallas import tpu_sc as plsc

_N = 256
_DT = 0.05
_DEPTH = 8
_BATCH = 262144
_L = 16
_NC = 2
_NS = 16
_NW = _NC * _NS
_BPW = _BATCH // _NW
_UNROLL = 2

_STEP = 0.390625
_DTB = 0.050048828125


def _rnbf16(a):
    u = plsc.bitcast(a, jnp.int32)
    u = (u + 0x7FFF + ((u >> 16) & 1)) & jnp.int32(-65536)
    return plsc.bitcast(u, jnp.float32)


def _nearest_idx(x):
    return (x * jnp.float32(_N) + jnp.float32(0.5)).astype(jnp.int32)


def _sc_body(x_hbm, f_hbm, out_hbm, buf, ftab):
    wid = lax.axis_index("s") * _NC + lax.axis_index("c")
    base = wid * (_BPW * 2)
    pltpu.sync_copy(f_hbm, ftab.at[pl.ds(0, _N)])
    pltpu.sync_copy(x_hbm.at[pl.ds(base, _BPW * 2)], buf)
    last = plsc.load_gather(ftab, [jnp.full((_L,), _N - 1, jnp.int32)])
    for j in range(_N // _L):
        ftab[pl.ds(_N + j * _L, _L)] = last

    dt = jnp.float32(_DT)

    @plsc.parallel_loop(0, _BPW // _L, unroll=_UNROLL)
    def _loop(i):
        off = (i >> 3) * 256 + (i & 7) * _L
        x = buf[pl.ds(off, _L)]
        v = buf[pl.ds(off + 128, _L)]
        v = _rnbf16(v)
        x = _rnbf16(x) + v * jnp.float32(_DTB)
        for _ in range(_DEPTH):
            idx = _nearest_idx(x)
            f = plsc.load_gather(ftab, [idx])
            v = v + dt * f
            x = x + v * dt
        buf[pl.ds(off, _L)] = x
        buf[pl.ds(off + 128, _L)] = v

    pltpu.sync_copy(buf, out_hbm.at[pl.ds(base, _BPW * 2)])


_mesh = plsc.VectorSubcoreMesh(core_axis_name="c", subcore_axis_name="s")

_sc_kernel = functools.partial(
    pl.kernel,
    mesh=_mesh,
    out_type=jax.ShapeDtypeStruct((_BATCH * 2,), jnp.float32),
    scratch_types=[
        pltpu.VMEM((_BPW * 2,), jnp.float32),
        pltpu.VMEM((2 * _N,), jnp.float32),
    ],
    compiler_params=pltpu.CompilerParams(needs_layout_passes=False),
)(_sc_body)


@jax.jit
def kernel(X, force):
    xf = X.reshape(_BATCH // 128, 128, 2).transpose(0, 2, 1).reshape(-1)
    out = _sc_kernel(xf, force)
    return (out.reshape(_BATCH // 128, 2, 128)
               .transpose(0, 2, 1).reshape(_BATCH, 2))
